# Initial kernel scaffold; baseline (speedup 1.0000x reference)
import jax
import jax.numpy as jnp
from jax.experimental import pallas as pl


def kernel(x):
    raise NotImplementedError("write your pallas kernel here")


# speedup vs baseline: 361.2881x; 361.2881x over previous
import functools

import jax
import jax.numpy as jnp
from jax import lax
from jax.experimental import pallas as pl
from jax.experimental.pallas import tpu as pltpu
from jax.experimental.pallas import tpu_sc as plsc

PRE = 1000
IOU_T = 0.55
POST = 300
NSLOT = 1024
NCHUNK = NSLOT // 16
NC, NS = 2, 16
NW = NC * NS
NCLS = 80
QMAX = -(-NCLS // NW)
OUTN = 304
NEG = -3.0e38


def _wbf_sc_body(x_hbm, buf_hbm, xv, idxl, created, mcoord, wacc, sacc,
                 cacc, rowv):
    wid = lax.axis_index("s") * NC + lax.axis_index("c")
    lanes = lax.broadcasted_iota(jnp.int32, (16,), 0)

    pltpu.sync_copy(x_hbm, xv)

    def _extract(chunk, lane):
        return jnp.sum(jnp.where(lanes == lane, chunk, chunk.dtype.type(0)))

    for q in range(QMAX):
        Q = q * NSLOT
        cid = wid + NW * q
        cf = cid.astype(jnp.float32)

        def part_body(ch, cnt, Q=Q, cf=cf):
            cls_chunk = xv[pl.ds(5 * NSLOT + ch * 16, 16)]
            m = cls_chunk == cf
            mi = jnp.where(m, 1, 0)
            pos = jnp.where(m, Q + cnt + plsc.cumsum(mi) - 1,
                            QMAX * NSLOT + lanes)
            plsc.store_scatter(idxl, [pos], lanes + ch * 16)
            return cnt + jnp.sum(mi)

        m_q = lax.fori_loop(0, NCHUNK, part_body, jnp.int32(0))

        def box_body(p, ncl, Q=Q):
            pb = (p >> 4) << 4
            plane = p & 15
            i = _extract(idxl[pl.ds(Q + pb, 16)], plane)
            ib = (i >> 4) << 4
            il = i & 15
            b0 = _extract(xv[pl.ds(0 * NSLOT + ib, 16)], il)
            b1 = _extract(xv[pl.ds(1 * NSLOT + ib, 16)], il)
            b2 = _extract(xv[pl.ds(2 * NSLOT + ib, 16)], il)
            b3 = _extract(xv[pl.ds(3 * NSLOT + ib, 16)], il)
            s = _extract(xv[pl.ds(4 * NSLOT + ib, 16)], il)
            a1 = (b2 - b0) * (b3 - b1)

            def ch_body(ch, jf, Q=Q):
                base = ch * 16
                m0 = mcoord[pl.ds(0 * QMAX * NSLOT + Q + base, 16)]
                m1 = mcoord[pl.ds(1 * QMAX * NSLOT + Q + base, 16)]
                m2 = mcoord[pl.ds(2 * QMAX * NSLOT + Q + base, 16)]
                m3 = mcoord[pl.ds(3 * QMAX * NSLOT + Q + base, 16)]
                iw = jnp.maximum(jnp.minimum(b2, m2) - jnp.maximum(b0, m0), 0.0)
                ih = jnp.maximum(jnp.minimum(b3, m3) - jnp.maximum(b1, m1), 0.0)
                inter = iw * ih
                a2 = (m2 - m0) * (m3 - m1)
                iou = inter / (a1 + a2 - inter)
                hit = (iou > IOU_T) & ((lanes + base) < ncl)
                pc = jnp.max(plsc.all_reduce_population_count(hit))
                ff = jnp.max(plsc.all_reduce_ffs(hit))
                return jnp.where((jf >= 16384) & (pc > 0), base + ff, jf)

            nch = (ncl + 15) >> 4
            jf = lax.fori_loop(0, nch, ch_body, jnp.int32(16384))
            anyh = jf < 16384
            k = jnp.where(anyh, jf, ncl)
            kb = (k >> 4) << 4
            kl = k & 15
            sel = lanes == kl
            old_ss = sacc[pl.ds(Q + kb, 16)]
            ssc = jnp.where(sel, jnp.where(anyh, old_ss + s, s), old_ss)
            sacc[pl.ds(Q + kb, 16)] = ssc
            old_c = cacc[pl.ds(Q + kb, 16)]
            cacc[pl.ds(Q + kb, 16)] = jnp.where(
                sel, jnp.where(anyh, old_c + 1.0, 1.0), old_c)
            for d, bd in enumerate((b0, b1, b2, b3)):
                W = d * QMAX * NSLOT + Q
                old_w = wacc[pl.ds(W + kb, 16)]
                wc = jnp.where(sel, jnp.where(anyh, old_w + s * bd, s * bd),
                               old_w)
                wacc[pl.ds(W + kb, 16)] = wc
                mcoord[pl.ds(W + kb, 16)] = jnp.where(
                    sel, wc / ssc, mcoord[pl.ds(W + kb, 16)])
            crv = jnp.where(anyh, jnp.int32(-1), ncl)
            created[pl.ds(Q + pb, 16)] = jnp.where(
                lanes == plane, crv, created[pl.ds(Q + pb, 16)])
            return ncl + jnp.where(anyh, 0, 1)

        lax.fori_loop(0, m_q, box_body, jnp.int32(0))

        def out_body(p, carry, Q=Q, cf=cf):
            pb = (p >> 4) << 4
            plane = p & 15
            i = _extract(idxl[pl.ds(Q + pb, 16)], plane)
            k = _extract(created[pl.ds(Q + pb, 16)], plane)
            isnew = k >= 0
            ku = jnp.maximum(k, 0)
            kb = (ku >> 4) << 4
            selk = lanes == (ku & 15)
            w0 = jnp.sum(jnp.where(selk, wacc[pl.ds(0 * QMAX * NSLOT + Q + kb, 16)], 0.0))
            w1 = jnp.sum(jnp.where(selk, wacc[pl.ds(1 * QMAX * NSLOT + Q + kb, 16)], 0.0))
            w2 = jnp.sum(jnp.where(selk, wacc[pl.ds(2 * QMAX * NSLOT + Q + kb, 16)], 0.0))
            w3 = jnp.sum(jnp.where(selk, wacc[pl.ds(3 * QMAX * NSLOT + Q + kb, 16)], 0.0))
            ssv = jnp.sum(jnp.where(selk, sacc[pl.ds(Q + kb, 16)], 0.0))
            cnv = jnp.sum(jnp.where(selk, cacc[pl.ds(Q + kb, 16)], 0.0))
            tbv = cf * 1024.0 + i.astype(jnp.float32)
            num = jnp.zeros((16,), jnp.float32)
            den = jnp.ones((16,), jnp.float32)
            keyn = jnp.where(isnew, ssv, NEG)
            keyd = jnp.where(isnew, cnv, 1.0)
            for li, (nv, dv) in enumerate((
                    (w0, ssv), (w1, ssv), (w2, ssv), (w3, ssv),
                    (ssv, cnv), (cf, 1.0), (keyn, keyd), (tbv, 1.0))):
                num = jnp.where(lanes == li, nv, num)
                den = jnp.where(lanes == li, dv, den)
            rowv[:] = num / den
            pltpu.sync_copy(rowv, buf_hbm.at[pl.ds(i * 16, 16)])
            return carry

        lax.fori_loop(0, m_q, out_body, jnp.int32(0))


def _rank_tc_body(buf_ref, buft_ref, o_ref):
    buft = buft_ref[:]
    jcol = lax.broadcasted_iota(jnp.int32, (NSLOT, 1), 0)
    jrow = lax.broadcasted_iota(jnp.int32, (1, NSLOT), 1)
    vcol = jcol < PRE
    vrow = jrow < PRE
    key_c = jnp.where(vcol, buf_ref[:, 6:7], NEG)
    tb_c = jnp.where(vcol, buf_ref[:, 7:8], 2.0e8 + jcol.astype(jnp.float32))
    key_r = jnp.where(vrow, buft[6:7, :], NEG)
    tb_r = jnp.where(vrow, buft[7:8, :], 2.0e8 + jrow.astype(jnp.float32))
    before = (key_c > key_r) | ((key_c == key_r) & (tb_c < tb_r))
    rank = jnp.sum(before.astype(jnp.float32), axis=0, keepdims=True)
    rsel = lax.broadcasted_iota(jnp.int32, (OUTN, 1), 0).astype(jnp.float32)
    onehot = (rank == rsel).astype(jnp.float32)
    bufc = jnp.where(vrow, buft, 0.0)
    o_ref[:] = lax.dot_general(
        onehot, bufc, (((1,), (1,)), ((), ())),
        precision=lax.Precision.HIGHEST,
        preferred_element_type=jnp.float32)


@jax.jit
def kernel(x):
    xt = x[:PRE].astype(jnp.float32)
    xsoa = jnp.full((8, NSLOT), -1.0, jnp.float32).at[:6, :PRE].set(xt.T)
    xsoa = xsoa.reshape(8 * NSLOT)

    mesh = plsc.VectorSubcoreMesh(core_axis_name="c", subcore_axis_name="s",
                                  num_cores=NC, num_subcores=NS)
    phase1 = pl.kernel(
        _wbf_sc_body,
        out_type=jax.ShapeDtypeStruct((NSLOT * 16,), jnp.float32),
        mesh=mesh,
        compiler_params=pltpu.CompilerParams(needs_layout_passes=False),
        scratch_types=[
            pltpu.VMEM((8 * NSLOT,), jnp.float32),
            pltpu.VMEM((QMAX * NSLOT + 16,), jnp.int32),
            pltpu.VMEM((QMAX * NSLOT,), jnp.int32),
            pltpu.VMEM((4 * QMAX * NSLOT,), jnp.float32),
            pltpu.VMEM((4 * QMAX * NSLOT,), jnp.float32),
            pltpu.VMEM((QMAX * NSLOT,), jnp.float32),
            pltpu.VMEM((QMAX * NSLOT,), jnp.float32),
            pltpu.VMEM((16,), jnp.float32),
        ],
    )
    buf = phase1(xsoa).reshape(NSLOT, 16)

    out = pl.pallas_call(
        _rank_tc_body,
        out_shape=jax.ShapeDtypeStruct((OUTN, 16), jnp.float32),
    )(buf, buf.T)
    return out[:POST, :6]


# speedup vs baseline: 436.2138x; 1.2074x over previous
import functools

import jax
import jax.numpy as jnp
from jax import lax
from jax.experimental import pallas as pl
from jax.experimental.pallas import tpu as pltpu
from jax.experimental.pallas import tpu_sc as plsc

PRE = 1000
IOU_T = 0.55
POST = 300
NSLOT = 1024
NCHUNK = NSLOT // 16
NC, NS = 2, 16
NW = NC * NS
NCLS = 80
QMAX = -(-NCLS // NW)
QN = QMAX * NSLOT
OUTN = 304
NEG = -3.0e38


def _wbf_sc_body(x_hbm, buf_hbm, xv, idxl, created, mcoord, wacc, sacc,
                 cacc, arena, rowv, sem):
    wid = lax.axis_index("s") * NC + lax.axis_index("c")
    lanes = lax.broadcasted_iota(jnp.int32, (16,), 0)

    pltpu.sync_copy(x_hbm, xv)

    cfs = [(wid + NW * q).astype(jnp.float32) for q in range(QMAX)]

    def part_body(ch, cnts):
        cls_chunk = xv[pl.ds(5 * NSLOT + ch * 16, 16)]
        idxs = lanes + ch * 16
        out = []
        for q in range(QMAX):
            m = cls_chunk == cfs[q]
            mi = jnp.where(m, 1, 0)
            pos = jnp.where(m, q * NSLOT + cnts[q] + plsc.cumsum(mi) - 1,
                            QN + lanes)
            plsc.store_scatter(idxl, [pos], idxs)
            out.append(cnts[q] + plsc.all_reduce_population_count(m))
        return tuple(out)

    zero = jnp.zeros((16,), jnp.int32)
    cnts = lax.fori_loop(0, NCHUNK, part_body, (zero,) * QMAX)
    ms = [jnp.max(c) for c in cnts]

    for q in range(QMAX):
        Q = q * NSLOT

        def box_body(p, ncl, Q=Q):
            psp = jnp.full((16,), p, jnp.int32)
            isp = plsc.load_gather(idxl, [Q + psp])
            b0 = plsc.load_gather(xv, [isp])
            b1 = plsc.load_gather(xv, [NSLOT + isp])
            b2 = plsc.load_gather(xv, [2 * NSLOT + isp])
            b3 = plsc.load_gather(xv, [3 * NSLOT + isp])
            s = plsc.load_gather(xv, [4 * NSLOT + isp])
            a1 = (b2 - b0) * (b3 - b1)

            def ch_body(ch, jf, Q=Q):
                base = ch * 16
                m0 = mcoord[pl.ds(0 * QN + Q + base, 16)]
                m1 = mcoord[pl.ds(1 * QN + Q + base, 16)]
                m2 = mcoord[pl.ds(2 * QN + Q + base, 16)]
                m3 = mcoord[pl.ds(3 * QN + Q + base, 16)]
                iw = jnp.maximum(jnp.minimum(b2, m2) - jnp.maximum(b0, m0), 0.0)
                ih = jnp.maximum(jnp.minimum(b3, m3) - jnp.maximum(b1, m1), 0.0)
                inter = iw * ih
                a2 = (m2 - m0) * (m3 - m1)
                iou = inter / (a1 + a2 - inter)
                hit = (iou > IOU_T) & ((lanes + base) < ncl)
                pc = plsc.all_reduce_population_count(hit)
                ff = plsc.all_reduce_ffs(hit)
                return jnp.where((jf >= 16384) & (pc > 0), base + ff, jf)

            nch = (ncl + 15) >> 4
            jf = lax.fori_loop(0, nch, ch_body,
                               jnp.full((16,), 16384, jnp.int32))
            anyv = jf < 16384
            anyh = jnp.max(jnp.where(anyv, 1, 0)) > 0
            ksp = jnp.where(anyv, jf, jnp.full((16,), ncl, jnp.int32))
            addr = Q + ksp
            old_ss = plsc.load_gather(sacc, [addr])
            ssc = jnp.where(anyv, old_ss + s, s)
            plsc.store_scatter(sacc, [addr], ssc)
            old_c = plsc.load_gather(cacc, [addr])
            plsc.store_scatter(cacc, [addr],
                               jnp.where(anyv, old_c + 1.0, 1.0))
            for d, bd in enumerate((b0, b1, b2, b3)):
                old_w = plsc.load_gather(wacc, [d * QN + addr])
                wc = jnp.where(anyv, old_w + s * bd, s * bd)
                plsc.store_scatter(wacc, [d * QN + addr], wc)
                plsc.store_scatter(mcoord, [d * QN + addr], wc / ssc)
            crv = jnp.where(anyv, jnp.full((16,), -1, jnp.int32),
                            jnp.full((16,), ncl, jnp.int32))
            plsc.store_scatter(created, [Q + psp], crv)
            return ncl + jnp.where(anyh, 0, 1)

        lax.fori_loop(0, ms[q], box_body, jnp.int32(0))

    nfired = jnp.int32(0)
    for q in range(QMAX):
        Q = q * NSLOT
        cf = cfs[q]

        def out_body(p, apos, Q=Q, cf=cf):
            psp = jnp.full((16,), p, jnp.int32)
            isp = plsc.load_gather(idxl, [Q + psp])
            ksp = plsc.load_gather(created, [Q + psp])
            isnew = ksp >= 0
            addr = Q + jnp.maximum(ksp, 0)
            w0 = plsc.load_gather(wacc, [0 * QN + addr])
            w1 = plsc.load_gather(wacc, [1 * QN + addr])
            w2 = plsc.load_gather(wacc, [2 * QN + addr])
            w3 = plsc.load_gather(wacc, [3 * QN + addr])
            ssv = plsc.load_gather(sacc, [addr])
            cnv = plsc.load_gather(cacc, [addr])
            tbv = cf * 1024.0 + isp.astype(jnp.float32)
            keyn = jnp.where(isnew, ssv, NEG)
            keyd = jnp.where(isnew, cnv, 1.0)
            num = jnp.zeros((16,), jnp.float32)
            den = jnp.ones((16,), jnp.float32)
            for li, (nv, dv) in enumerate((
                    (w0, ssv), (w1, ssv), (w2, ssv), (w3, ssv),
                    (ssv, cnv), (cf, 1.0), (keyn, keyd), (tbv, 1.0))):
                num = jnp.where(lanes == li, nv, num)
                den = jnp.where(lanes == li, dv, den)
            arena[pl.ds(apos * 16, 16)] = num / den
            i_sc = jnp.max(isp)
            pltpu.async_copy(arena.at[pl.ds(apos * 16, 16)],
                             buf_hbm.at[pl.ds(i_sc * 16, 16)], sem)
            return apos + 1

        nfired = lax.fori_loop(0, ms[q], out_body, nfired)

    def drain_body(p, c):
        pltpu.make_async_copy(x_hbm.at[pl.ds(0, 16)], rowv, sem).wait()
        return c

    lax.fori_loop(0, nfired, drain_body, jnp.int32(0))


def _rank_tc_body(buf_ref, buft_ref, o_ref):
    buft = buft_ref[:]
    jcol = lax.broadcasted_iota(jnp.int32, (NSLOT, 1), 0)
    jrow = lax.broadcasted_iota(jnp.int32, (1, NSLOT), 1)
    vcol = jcol < PRE
    vrow = jrow < PRE
    key_c = jnp.where(vcol, buf_ref[:, 6:7], NEG)
    tb_c = jnp.where(vcol, buf_ref[:, 7:8], 2.0e8 + jcol.astype(jnp.float32))
    key_r = jnp.where(vrow, buft[6:7, :], NEG)
    tb_r = jnp.where(vrow, buft[7:8, :], 2.0e8 + jrow.astype(jnp.float32))
    before = (key_c > key_r) | ((key_c == key_r) & (tb_c < tb_r))
    rank = jnp.sum(before.astype(jnp.float32), axis=0, keepdims=True)
    rsel = lax.broadcasted_iota(jnp.int32, (OUTN, 1), 0).astype(jnp.float32)
    onehot = (rank == rsel).astype(jnp.float32)
    bufc = jnp.where(vrow, buft, 0.0)
    o_ref[:] = lax.dot_general(
        onehot, bufc, (((1,), (1,)), ((), ())),
        precision=lax.Precision.HIGHEST,
        preferred_element_type=jnp.float32)


@jax.jit
def kernel(x):
    xt = x[:PRE].astype(jnp.float32)
    xsoa = jnp.full((8, NSLOT), -1.0, jnp.float32).at[:6, :PRE].set(xt.T)
    xsoa = xsoa.reshape(8 * NSLOT)

    mesh = plsc.VectorSubcoreMesh(core_axis_name="c", subcore_axis_name="s",
                                  num_cores=NC, num_subcores=NS)
    phase1 = pl.kernel(
        _wbf_sc_body,
        out_type=jax.ShapeDtypeStruct((NSLOT * 16,), jnp.float32),
        mesh=mesh,
        compiler_params=pltpu.CompilerParams(needs_layout_passes=False),
        scratch_types=[
            pltpu.VMEM((8 * NSLOT,), jnp.float32),
            pltpu.VMEM((QN + 16,), jnp.int32),
            pltpu.VMEM((QN,), jnp.int32),
            pltpu.VMEM((4 * QN,), jnp.float32),
            pltpu.VMEM((4 * QN,), jnp.float32),
            pltpu.VMEM((QN,), jnp.float32),
            pltpu.VMEM((QN,), jnp.float32),
            pltpu.VMEM((NSLOT * 16,), jnp.float32),
            pltpu.VMEM((16,), jnp.float32),
            pltpu.SemaphoreType.DMA,
        ],
    )
    buf = phase1(xsoa).reshape(NSLOT, 16)

    out = pl.pallas_call(
        _rank_tc_body,
        out_shape=jax.ShapeDtypeStruct((OUTN, 16), jnp.float32),
    )(buf, buf.T)
    return out[:POST, :6]
